# baseline (speedup 1.0000x reference)
import functools

import jax
import jax.numpy as jnp
from jax import lax
from jax.experimental import pallas as pl
import jax.experimental.pallas.tpu as pltpu
from jax.experimental.pallas import tpu_sc as plsc

_ROWS = 5000
_NG = 64
_CHUNK = 400
_NW = 25


def _layernorm(v, eps=1e-5):
    mu = jnp.mean(v, axis=-1, keepdims=True)
    var = jnp.mean((v - mu) ** 2, axis=-1, keepdims=True)
    return (v - mu) * jax.lax.rsqrt(var + eps)


def _gelu_exact(v):
    return 0.5 * v * (1.0 + jax.lax.erf(v * (2.0 ** -0.5)))


def _embed_kernel(x_ref, pe_ref, w_in_ref, b_in_ref, g_in_ref, bt_in_ref,
                  h_ref):
    h = jnp.dot(x_ref[...], w_in_ref[...], preferred_element_type=jnp.float32)
    h = _layernorm(h + b_in_ref[...]) * g_in_ref[...] + bt_in_ref[...]
    pe = pe_ref[...]
    h_ref[...] = h + jnp.concatenate([pe] * (_ROWS // 1000), axis=0)


def _sc_pool(h_hbm, batch_hbm, zeros_hbm, out_hbm, idx_v, rows_v, acc_sh):
    c = lax.axis_index("c")
    s = lax.axis_index("s")
    w = c * 16 + s

    @pl.when(s == 0)
    def _init():
        pltpu.sync_copy(zeros_hbm, acc_sh)

    plsc.subcore_barrier()

    @pl.when(w < _NW)
    def _accum():
        base = w * _CHUNK
        pltpu.sync_copy(batch_hbm.at[pl.ds(base, _CHUNK)], idx_v)
        pltpu.sync_copy(h_hbm.at[pl.ds(base, _CHUNK)], rows_v)
        pltpu.sync_copy(rows_v, acc_sh.at[idx_v], add=True)

    plsc.subcore_barrier()

    @pl.when(s == 0)
    def _flush():
        pltpu.sync_copy(acc_sh, out_hbm.at[c])


def _clf_kernel(parts_ref, c1w_ref, rw1_ref, rw2_ref, c1b_ref, c1g_ref,
                c1bt_ref, rg_ref, rbt_ref, rb1_ref, rb2_ref, c2w_ref,
                c2b_ref, out_ref):
    pooled = parts_ref[0] + parts_ref[1]
    c = jnp.dot(pooled, c1w_ref[...], preferred_element_type=jnp.float32)
    c = _layernorm(c + c1b_ref[...]) * c1g_ref[...] + c1bt_ref[...]
    c = _gelu_exact(c)
    t = _layernorm(c) * rg_ref[...] + rbt_ref[...]
    inner = _gelu_exact(
        jnp.dot(t, rw1_ref[...], preferred_element_type=jnp.float32)
        + rb1_ref[...])
    r = c + jnp.dot(inner, rw2_ref[...],
                    preferred_element_type=jnp.float32) + rb2_ref[...]
    o = jnp.sum(r * c2w_ref[...], axis=-1, keepdims=True) + c2b_ref[0, 0]
    out_ref[...] = jnp.broadcast_to(o, (_NG, 128))


@jax.jit
def kernel(x, edge_index, edge_attr, batch, params):
    p = params
    n, d = x.shape
    nsteps = n // _ROWS

    def row(v):
        return v.reshape(1, -1)

    vec_spec = pl.BlockSpec((1, d), lambda *_: (0, 0))

    h = pl.pallas_call(
        _embed_kernel,
        grid=(nsteps,),
        in_specs=[
            pl.BlockSpec((_ROWS, d), lambda i: (i, 0)),
            pl.BlockSpec((1000, d), lambda i: (0, 0)),
            pl.BlockSpec((d, d), lambda i: (0, 0)),
        ] + [vec_spec] * 3,
        out_specs=pl.BlockSpec((_ROWS, d), lambda i: (i, 0)),
        out_shape=jax.ShapeDtypeStruct((n, d), jnp.float32),
    )(x, p['pe'], p['W_in'], row(p['b_in']), row(p['g_in']),
      row(p['bt_in']))

    mesh = plsc.VectorSubcoreMesh(core_axis_name="c", subcore_axis_name="s")
    pool_fn = functools.partial(
        pl.kernel, mesh=mesh,
        out_type=jax.ShapeDtypeStruct((2, _NG, 128), jnp.float32),
        scratch_types=[
            pltpu.VMEM((_CHUNK,), jnp.int32),
            pltpu.VMEM((_CHUNK, 128), jnp.float32),
            pltpu.VMEM_SHARED((_NG, 128), jnp.float32),
        ],
    )(_sc_pool)
    parts = pool_fn(h, batch, jnp.zeros((_NG, 128), jnp.float32))

    full = pl.pallas_call(
        _clf_kernel,
        grid=(1,),
        in_specs=[
            pl.BlockSpec((2, _NG, 128), lambda i: (0, 0, 0)),
            pl.BlockSpec((d, 128), lambda i: (0, 0)),
            pl.BlockSpec((128, 128), lambda i: (0, 0)),
            pl.BlockSpec((128, 128), lambda i: (0, 0)),
        ] + [pl.BlockSpec((1, d), lambda i: (0, 0))] * 9,
        out_specs=pl.BlockSpec((_NG, 128), lambda i: (0, 0)),
        out_shape=jax.ShapeDtypeStruct((_NG, 128), jnp.float32),
    )(parts, p['c1W'], p['rW1'], p['rW2'], row(p['c1b']), row(p['c1g']),
      row(p['c1bt']), row(p['rg']), row(p['rbt']), row(p['rb1']),
      row(p['rb2']), row(p['c2W'][:, 0]),
      jnp.broadcast_to(p['c2b'], (1, d)))

    return full[:, :1]


# speedup vs baseline: 2.8413x; 2.8413x over previous
import functools

import jax
import jax.numpy as jnp
from jax.experimental import pallas as pl
import jax.experimental.pallas.tpu as pltpu

_ROWS = 10000
_NG = 64


def _layernorm(v, eps=1e-5):
    mu = jnp.mean(v, axis=-1, keepdims=True)
    var = jnp.mean((v - mu) ** 2, axis=-1, keepdims=True)
    return (v - mu) * jax.lax.rsqrt(var + eps)


def _gelu_exact(v):
    return 0.5 * v * (1.0 + jax.lax.erf(v * (2.0 ** -0.5)))


def _fused_kernel(x_ref, pe_ref, batch_ref, w_in_ref, c1w_ref, rw1_ref,
                  rw2_ref, b_in_ref, g_in_ref, bt_in_ref, c1b_ref, c1g_ref,
                  c1bt_ref, rg_ref, rbt_ref, rb1_ref, rb2_ref, c2w_ref,
                  c2b_ref, out_ref, acc_ref, *, nsteps):
    i = pl.program_id(0)

    h = jnp.dot(x_ref[...], w_in_ref[...], preferred_element_type=jnp.float32)
    h = _layernorm(h + b_in_ref[...]) * g_in_ref[...] + bt_in_ref[...]
    pe = pe_ref[...]
    h = h + jnp.concatenate([pe] * (_ROWS // 1000), axis=0)

    seg = batch_ref[0, 0, :]
    onehot = (jax.lax.broadcasted_iota(jnp.int32, (_NG, _ROWS), 0)
              == seg[None, :]).astype(jnp.float32)
    h_hi = h.astype(jnp.bfloat16).astype(jnp.float32)
    h_lo = h - h_hi
    part = (jnp.dot(onehot, h_hi, preferred_element_type=jnp.float32)
            + jnp.dot(onehot, h_lo, preferred_element_type=jnp.float32))

    @pl.when(i == 0)
    def _init():
        acc_ref[...] = part

    @pl.when(i > 0)
    def _accum():
        acc_ref[...] += part

    @pl.when(i == nsteps - 1)
    def _classifier():
        pooled = acc_ref[...]
        c = jnp.dot(pooled, c1w_ref[...], preferred_element_type=jnp.float32)
        c = _layernorm(c + c1b_ref[...]) * c1g_ref[...] + c1bt_ref[...]
        c = _gelu_exact(c)
        t = _layernorm(c) * rg_ref[...] + rbt_ref[...]
        inner = _gelu_exact(
            jnp.dot(t, rw1_ref[...], preferred_element_type=jnp.float32)
            + rb1_ref[...])
        r = c + jnp.dot(inner, rw2_ref[...],
                        preferred_element_type=jnp.float32) + rb2_ref[...]
        o = jnp.sum(r * c2w_ref[...], axis=-1, keepdims=True) + c2b_ref[0, 0]
        out_ref[...] = jnp.broadcast_to(o, (_NG, 128))


@jax.jit
def kernel(x, edge_index, edge_attr, batch, params):
    p = params
    n, d = x.shape
    nsteps = n // _ROWS

    batch3 = batch.reshape(nsteps, 1, _ROWS)

    def row(v):
        return v.reshape(1, -1)

    vec_spec = pl.BlockSpec((1, d), lambda i: (0, 0))
    full = pl.pallas_call(
        functools.partial(_fused_kernel, nsteps=nsteps),
        grid=(nsteps,),
        in_specs=[
            pl.BlockSpec((_ROWS, d), lambda i: (i, 0)),
            pl.BlockSpec((1000, d), lambda i: (0, 0)),
            pl.BlockSpec((1, 1, _ROWS), lambda i: (i, 0, 0)),
            pl.BlockSpec((d, d), lambda i: (0, 0)),
            pl.BlockSpec((d, 128), lambda i: (0, 0)),
            pl.BlockSpec((128, 128), lambda i: (0, 0)),
            pl.BlockSpec((128, 128), lambda i: (0, 0)),
        ] + [vec_spec] * 12,
        out_specs=pl.BlockSpec((_NG, 128), lambda i: (0, 0)),
        out_shape=jax.ShapeDtypeStruct((_NG, 128), jnp.float32),
        scratch_shapes=[pltpu.VMEM((_NG, 128), jnp.float32)],
    )(x, p['pe'], batch3, p['W_in'], p['c1W'], p['rW1'], p['rW2'],
      row(p['b_in']), row(p['g_in']), row(p['bt_in']), row(p['c1b']),
      row(p['c1g']), row(p['c1bt']), row(p['rg']), row(p['rbt']),
      row(p['rb1']), row(p['rb2']), row(p['c2W'][:, 0]),
      jnp.broadcast_to(p['c2b'], (1, d)))

    return full[:, :1]


# speedup vs baseline: 3.0210x; 1.0632x over previous
import functools

import jax
import jax.numpy as jnp
from jax.experimental import pallas as pl
import jax.experimental.pallas.tpu as pltpu

_ROWS = 10000
_NG = 64


def _layernorm(v, eps=1e-5):
    mu = jnp.mean(v, axis=-1, keepdims=True)
    var = jnp.mean((v - mu) ** 2, axis=-1, keepdims=True)
    return (v - mu) * jax.lax.rsqrt(var + eps)


def _gelu_exact(v):
    return 0.5 * v * (1.0 + jax.lax.erf(v * (2.0 ** -0.5)))


def _fused_kernel(x_ref, pe_ref, batch_ref, w_in_ref, c1w_ref, rw1_ref,
                  rw2_ref, b_in_ref, g_in_ref, bt_in_ref, c1b_ref, c1g_ref,
                  c1bt_ref, rg_ref, rbt_ref, rb1_ref, rb2_ref, c2w_ref,
                  c2b_ref, out_ref, acc_ref, *, nsteps):
    i = pl.program_id(0)

    mscale = jnp.full((128, 128), 1.0 / 128, jnp.float32)
    hb = (jnp.dot(x_ref[...], w_in_ref[...],
                  preferred_element_type=jnp.float32) + b_in_ref[...])
    mu = jnp.dot(hb, mscale, preferred_element_type=jnp.float32)
    e = hb - mu
    var = jnp.mean(e * e, axis=-1, keepdims=True)
    h = e * jax.lax.rsqrt(var + 1e-5) * g_in_ref[...] + bt_in_ref[...]
    pe = pe_ref[...]
    h = h + jnp.concatenate([pe] * (_ROWS // 1000), axis=0)

    seg = batch_ref[0, 0, :]
    onehot = (jax.lax.broadcasted_iota(jnp.int32, (_NG, _ROWS), 0)
              == seg[None, :]).astype(jnp.float32)
    h_hi = h.astype(jnp.bfloat16).astype(jnp.float32)
    h_lo = h - h_hi
    part = (jnp.dot(onehot, h_hi, preferred_element_type=jnp.float32)
            + jnp.dot(onehot, h_lo, preferred_element_type=jnp.float32))

    @pl.when(i == 0)
    def _init():
        acc_ref[...] = part

    @pl.when(i > 0)
    def _accum():
        acc_ref[...] += part

    @pl.when(i == nsteps - 1)
    def _classifier():
        pooled = acc_ref[...]
        c = jnp.dot(pooled, c1w_ref[...], preferred_element_type=jnp.float32)
        c = _layernorm(c + c1b_ref[...]) * c1g_ref[...] + c1bt_ref[...]
        c = _gelu_exact(c)
        t = _layernorm(c) * rg_ref[...] + rbt_ref[...]
        inner = _gelu_exact(
            jnp.dot(t, rw1_ref[...], preferred_element_type=jnp.float32)
            + rb1_ref[...])
        r = c + jnp.dot(inner, rw2_ref[...],
                        preferred_element_type=jnp.float32) + rb2_ref[...]
        o = jnp.sum(r * c2w_ref[...], axis=-1, keepdims=True) + c2b_ref[0, 0]
        out_ref[...] = jnp.broadcast_to(o, (_NG, 128))


@jax.jit
def kernel(x, edge_index, edge_attr, batch, params):
    p = params
    n, d = x.shape
    nsteps = n // _ROWS

    batch3 = batch.reshape(nsteps, 1, _ROWS)

    def row(v):
        return v.reshape(1, -1)

    vec_spec = pl.BlockSpec((1, d), lambda i: (0, 0))
    full = pl.pallas_call(
        functools.partial(_fused_kernel, nsteps=nsteps),
        grid=(nsteps,),
        in_specs=[
            pl.BlockSpec((_ROWS, d), lambda i: (i, 0)),
            pl.BlockSpec((1000, d), lambda i: (0, 0)),
            pl.BlockSpec((1, 1, _ROWS), lambda i: (i, 0, 0)),
            pl.BlockSpec((d, d), lambda i: (0, 0)),
            pl.BlockSpec((d, 128), lambda i: (0, 0)),
            pl.BlockSpec((128, 128), lambda i: (0, 0)),
            pl.BlockSpec((128, 128), lambda i: (0, 0)),
        ] + [vec_spec] * 12,
        out_specs=pl.BlockSpec((_NG, 128), lambda i: (0, 0)),
        out_shape=jax.ShapeDtypeStruct((_NG, 128), jnp.float32),
        scratch_shapes=[pltpu.VMEM((_NG, 128), jnp.float32)],
    )(x, p['pe'], batch3, p['W_in'], p['c1W'], p['rW1'], p['rW2'],
      row(p['b_in']), row(p['g_in']), row(p['bt_in']), row(p['c1b']),
      row(p['c1g']), row(p['c1bt']), row(p['rg']), row(p['rbt']),
      row(p['rb1']), row(p['rb2']), row(p['c2W'][:, 0]),
      jnp.broadcast_to(p['c2b'], (1, d)))

    return full[:, :1]
